# Initial kernel scaffold; baseline (speedup 1.0000x reference)
import jax
import jax.numpy as jnp
from jax.experimental import pallas as pl


def kernel(input, edge_index, edge_weight, weight, bias):
    raise NotImplementedError("write your pallas kernel here")


# speedup vs baseline: 3.9955x; 3.9955x over previous
import jax
import jax.numpy as jnp
from jax import lax
from jax.experimental import pallas as pl
from jax.experimental.pallas import tpu as pltpu
from jax.experimental.pallas import tpu_sc as plsc

N = 10000
D = 128
E = 320000
NC = 2
NS = 16
LANES = 16
CHUNK = 80
EDGES_PER_TILE = E // (NC * NS)
N_PER_TILE = N // NS


def _agg_body(x_hbm, col_hbm, row_hbm, w_hbm, out_hbm,
              col_v, row_v, w_v, rows_v, agg, sem):
    c = lax.axis_index("c")
    s = lax.axis_index("s")

    def zero_row(r, carry):
        for j in range(D // LANES):
            rows_v[r, pl.ds(j * LANES, LANES)] = jnp.zeros((LANES,), jnp.float32)
        return carry
    lax.fori_loop(0, CHUNK, zero_row, 0)
    for k in range(N_PER_TILE // CHUNK):
        pltpu.sync_copy(rows_v, agg.at[pl.ds(s * N_PER_TILE + k * CHUNK, CHUNK)])
    rem = N_PER_TILE % CHUNK
    if rem:
        pltpu.sync_copy(
            rows_v.at[pl.ds(0, rem)],
            agg.at[pl.ds(s * N_PER_TILE + (N_PER_TILE // CHUNK) * CHUNK, rem)])
    plsc.subcore_barrier()

    ebase = (c * NS + s) * EDGES_PER_TILE

    def step(i, carry):
        base = ebase + i * CHUNK
        pltpu.sync_copy(col_hbm.at[pl.ds(base, CHUNK)], col_v)
        pltpu.sync_copy(row_hbm.at[pl.ds(base, CHUNK)], row_v)
        pltpu.sync_copy(w_hbm.at[pl.ds(base, CHUNK)], w_v)
        pltpu.async_copy(x_hbm.at[col_v], rows_v, sem).wait()

        def scale(e, inner):
            wsplat = plsc.load_gather(
                w_v, [jnp.full((LANES,), e, dtype=jnp.int32)])
            for j in range(D // LANES):
                sl = pl.ds(j * LANES, LANES)
                rows_v[e, sl] = rows_v[e, sl] * wsplat
            return inner
        lax.fori_loop(0, CHUNK, scale, 0)

        pltpu.sync_copy(rows_v, agg.at[row_v], add=True)
        return carry
    lax.fori_loop(0, EDGES_PER_TILE // CHUNK, step, 0)

    plsc.subcore_barrier()
    pltpu.sync_copy(agg.at[pl.ds(s * N_PER_TILE, N_PER_TILE)],
                    out_hbm.at[c].at[pl.ds(s * N_PER_TILE, N_PER_TILE)])


_agg = pl.kernel(
    _agg_body,
    out_type=jax.ShapeDtypeStruct((NC, N, D), jnp.float32),
    mesh=plsc.VectorSubcoreMesh(core_axis_name="c", subcore_axis_name="s"),
    compiler_params=pltpu.CompilerParams(use_tc_tiling_on_sc=False,
                                         needs_layout_passes=False),
    scratch_types=[
        pltpu.VMEM((CHUNK,), jnp.int32),
        pltpu.VMEM((CHUNK,), jnp.int32),
        pltpu.VMEM((CHUNK,), jnp.float32),
        pltpu.VMEM((CHUNK, D), jnp.float32),
        pltpu.VMEM_SHARED((N, D), jnp.float32),
        pltpu.SemaphoreType.DMA,
    ],
)


def _mm_body(x_ref, pa_ref, pb_ref, w1_ref, w2_ref, b_ref, o_ref):
    agg = pa_ref[...] + pb_ref[...]
    o_ref[...] = (
        jnp.dot(x_ref[...], w1_ref[...], preferred_element_type=jnp.float32)
        + jnp.dot(agg, w2_ref[...], preferred_element_type=jnp.float32)
        + b_ref[...]
    )


def kernel(input, edge_index, edge_weight, weight, bias):
    x = input
    row = edge_index[0]
    col = edge_index[1]
    partials = _agg(x, col, row, edge_weight)

    w1 = weight[:D]
    w2 = weight[D:]
    BLK = 1000
    out = pl.pallas_call(
        _mm_body,
        grid=(N // BLK,),
        in_specs=[
            pl.BlockSpec((BLK, D), lambda i: (i, 0)),
            pl.BlockSpec((BLK, D), lambda i: (i, 0)),
            pl.BlockSpec((BLK, D), lambda i: (i, 0)),
            pl.BlockSpec((D, D), lambda i: (0, 0)),
            pl.BlockSpec((D, D), lambda i: (0, 0)),
            pl.BlockSpec((1, D), lambda i: (0, 0)),
        ],
        out_specs=pl.BlockSpec((BLK, D), lambda i: (i, 0)),
        out_shape=jax.ShapeDtypeStruct((N, D), jnp.float32),
    )(x, partials[0], partials[1], w1, w2, bias.reshape(1, D))
    return out


# speedup vs baseline: 9.4887x; 2.3749x over previous
import jax
import jax.numpy as jnp
from jax import lax
from jax.experimental import pallas as pl
from jax.experimental.pallas import tpu as pltpu
from jax.experimental.pallas import tpu_sc as plsc

N = 10000
D = 128
E = 320000
NC = 2
NS = 16
LANES = 16
CHUNK = 80
UNROLL = 4
EDGES_PER_TILE = E // (NC * NS)
CHUNKS_PER_TILE = EDGES_PER_TILE // CHUNK
N_PER_TILE = N // NS


def _agg_body(x_hbm, col_hbm, row_hbm, w_hbm, out_hbm,
              colb, rowb, wb, rows0, rows1, agg, sem0, sem1):
    c = lax.axis_index("c")
    s = lax.axis_index("s")
    wid = c * NS + s
    cbase = wid * CHUNKS_PER_TILE

    pltpu.sync_copy(col_hbm.at[pl.ds(cbase, CHUNKS_PER_TILE)], colb)
    pltpu.sync_copy(row_hbm.at[pl.ds(cbase, CHUNKS_PER_TILE)], rowb)
    pltpu.sync_copy(w_hbm.at[pl.ds(cbase, CHUNKS_PER_TILE)], wb)

    def zero_row(r, carry):
        for j in range(D // LANES):
            rows0[r, pl.ds(j * LANES, LANES)] = jnp.zeros((LANES,), jnp.float32)
        return carry
    lax.fori_loop(0, CHUNK, zero_row, 0)
    for k in range(N_PER_TILE // CHUNK):
        pltpu.sync_copy(rows0, agg.at[pl.ds(s * N_PER_TILE + k * CHUNK, CHUNK)])
    rem = N_PER_TILE % CHUNK
    if rem:
        pltpu.sync_copy(
            rows0.at[pl.ds(0, rem)],
            agg.at[pl.ds(s * N_PER_TILE + (N_PER_TILE // CHUNK) * CHUNK, rem)])
    plsc.subcore_barrier()

    def start_gather(chunk, buf, sem):
        pltpu.async_copy(x_hbm.at[colb.at[chunk]], buf, sem)

    def wait_gather(chunk, buf, sem):
        pltpu.make_async_copy(x_hbm.at[colb.at[chunk]], buf, sem).wait()

    def process(chunk, buf):
        def scale(i, carry):
            e0 = i * UNROLL
            for u in range(UNROLL):
                e = e0 + u
                wsplat = plsc.load_gather(
                    wb, [jnp.full((LANES,), chunk, dtype=jnp.int32),
                         jnp.full((LANES,), e, dtype=jnp.int32)])
                for j in range(D // LANES):
                    sl = pl.ds(j * LANES, LANES)
                    buf[e, sl] = buf[e, sl] * wsplat
            return carry
        lax.fori_loop(0, CHUNK // UNROLL, scale, 0)
        pltpu.sync_copy(buf, agg.at[rowb.at[chunk]], add=True)

    start_gather(0, rows0, sem0)

    def pair(k, carry):
        c0 = 2 * k
        start_gather(c0 + 1, rows1, sem1)
        wait_gather(c0, rows0, sem0)
        process(c0, rows0)
        start_gather(c0 + 2, rows0, sem0)
        wait_gather(c0 + 1, rows1, sem1)
        process(c0 + 1, rows1)
        return carry
    lax.fori_loop(0, (CHUNKS_PER_TILE - 1) // 2, pair, 0)

    last = CHUNKS_PER_TILE - 1
    wait_gather(last, rows0, sem0)
    process(last, rows0)

    plsc.subcore_barrier()
    pltpu.sync_copy(agg.at[pl.ds(s * N_PER_TILE, N_PER_TILE)],
                    out_hbm.at[c].at[pl.ds(s * N_PER_TILE, N_PER_TILE)])


_agg = pl.kernel(
    _agg_body,
    out_type=jax.ShapeDtypeStruct((NC, N, D), jnp.float32),
    mesh=plsc.VectorSubcoreMesh(core_axis_name="c", subcore_axis_name="s"),
    compiler_params=pltpu.CompilerParams(use_tc_tiling_on_sc=False,
                                         needs_layout_passes=False),
    scratch_types=[
        pltpu.VMEM((CHUNKS_PER_TILE, CHUNK), jnp.int32),
        pltpu.VMEM((CHUNKS_PER_TILE, CHUNK), jnp.int32),
        pltpu.VMEM((CHUNKS_PER_TILE, CHUNK), jnp.float32),
        pltpu.VMEM((CHUNK, D), jnp.float32),
        pltpu.VMEM((CHUNK, D), jnp.float32),
        pltpu.VMEM_SHARED((N, D), jnp.float32),
        pltpu.SemaphoreType.DMA,
        pltpu.SemaphoreType.DMA,
    ],
)


def _mm_body(x_ref, pa_ref, pb_ref, w1_ref, w2_ref, b_ref, o_ref):
    agg = pa_ref[...] + pb_ref[...]
    o_ref[...] = (
        jnp.dot(x_ref[...], w1_ref[...], preferred_element_type=jnp.float32)
        + jnp.dot(agg, w2_ref[...], preferred_element_type=jnp.float32)
        + b_ref[...]
    )


def kernel(input, edge_index, edge_weight, weight, bias):
    x = input
    row2d = edge_index[0].reshape(E // CHUNK, CHUNK)
    col2d = edge_index[1].reshape(E // CHUNK, CHUNK)
    w2d = edge_weight.reshape(E // CHUNK, CHUNK)
    partials = _agg(x, col2d, row2d, w2d)

    w1 = weight[:D]
    w2 = weight[D:]
    BLK = 1000
    out = pl.pallas_call(
        _mm_body,
        grid=(N // BLK,),
        in_specs=[
            pl.BlockSpec((BLK, D), lambda i: (i, 0)),
            pl.BlockSpec((BLK, D), lambda i: (i, 0)),
            pl.BlockSpec((BLK, D), lambda i: (i, 0)),
            pl.BlockSpec((D, D), lambda i: (0, 0)),
            pl.BlockSpec((D, D), lambda i: (0, 0)),
            pl.BlockSpec((1, D), lambda i: (0, 0)),
        ],
        out_specs=pl.BlockSpec((BLK, D), lambda i: (i, 0)),
        out_shape=jax.ShapeDtypeStruct((N, D), jnp.float32),
    )(x, partials[0], partials[1], w1, w2, bias.reshape(1, D))
    return out
